# Initial kernel scaffold; baseline (speedup 1.0000x reference)
import jax
import jax.numpy as jnp
from jax.experimental import pallas as pl


def kernel(hidden_states, gate_w, w_gate, w_up, w_down):
    raise NotImplementedError("write your pallas kernel here")


# speedup vs baseline: 1.8715x; 1.8715x over previous
import functools

import jax
import jax.numpy as jnp
from jax.experimental import pallas as pl


def _gate_kernel(x_ref, gwt_ref, pk_ref):
    x = x_ref[...]
    logits = jax.lax.dot_general(
        x, gwt_ref[...], (((1,), (0,)), ((), ())),
        preferred_element_type=jnp.float32)
    t, e = logits.shape
    eidx = jax.lax.broadcasted_iota(jnp.int32, (t, e), 1)
    m1 = jnp.max(logits, axis=1, keepdims=True)
    i1 = jnp.min(jnp.where(logits == m1, eidx, e), axis=1, keepdims=True)
    lm = jnp.where(eidx == i1, -1e30, logits)
    m2 = jnp.max(lm, axis=1, keepdims=True)
    i2 = jnp.min(jnp.where(lm == m2, eidx, e), axis=1, keepdims=True)
    w1 = 1.0 / (1.0 + jnp.exp(m2 - m1))
    pk_ref[...] = jnp.concatenate(
        [i1.astype(jnp.float32), i2.astype(jnp.float32), w1, 1.0 - w1], axis=1)


def _expert_kernel(xb_ref, wg_ref, wu_ref, wd_ref, pk_ref, out_ref, *, chunk):
    e = pl.program_id(1)
    ef = e.astype(jnp.float32)
    tb = xb_ref.shape[0]
    wg = wg_ref[0]
    wu = wu_ref[0]
    wd = wd_ref[0]
    for c in range(tb // chunk):
        sl = pl.ds(c * chunk, chunk)
        xc = xb_ref[sl, :]
        g = jax.lax.dot_general(xc, wg, (((1,), (0,)), ((), ())),
                                preferred_element_type=jnp.float32)
        u = jax.lax.dot_general(xc, wu, (((1,), (0,)), ((), ())),
                                preferred_element_type=jnp.float32)
        h = (g * jax.lax.logistic(g) * u).astype(jnp.bfloat16)
        y = jax.lax.dot_general(h, wd, (((1,), (0,)), ((), ())),
                                preferred_element_type=jnp.float32)
        w = (jnp.where(pk_ref[sl, 0:1] == ef, pk_ref[sl, 2:3], 0.0)
             + jnp.where(pk_ref[sl, 1:2] == ef, pk_ref[sl, 3:4], 0.0))
        contrib = y * w

        @pl.when(e == 0)
        def _():
            out_ref[sl, :] = contrib

        @pl.when(e != 0)
        def _():
            out_ref[sl, :] = out_ref[sl, :] + contrib


def _moe(x, gwt, wgb, wub, wdb, *, t_blocks, chunk, interpret=False):
    t, h = x.shape
    f = wgb.shape[2]
    e = gwt.shape[1]
    pk = pl.pallas_call(
        _gate_kernel,
        out_shape=jax.ShapeDtypeStruct((t, 4), jnp.float32),
        interpret=interpret,
    )(x, gwt)

    tb = t // t_blocks
    xb = x.astype(jnp.bfloat16)
    out = pl.pallas_call(
        functools.partial(_expert_kernel, chunk=chunk),
        grid=(t_blocks, e),
        in_specs=[
            pl.BlockSpec((tb, h), lambda i, j: (i, 0)),
            pl.BlockSpec((1, h, f), lambda i, j: (j, 0, 0)),
            pl.BlockSpec((1, h, f), lambda i, j: (j, 0, 0)),
            pl.BlockSpec((1, f, h), lambda i, j: (j, 0, 0)),
            pl.BlockSpec((tb, 4), lambda i, j: (i, 0)),
        ],
        out_specs=pl.BlockSpec((tb, h), lambda i, j: (i, 0)),
        out_shape=jax.ShapeDtypeStruct((t, h), jnp.float32),
        interpret=interpret,
    )(xb, wgb, wub, wdb, pk)
    return out


def kernel(hidden_states, gate_w, w_gate, w_up, w_down):
    b, s, h = hidden_states.shape
    x = hidden_states.reshape(b * s, h)
    out = _moe(
        x,
        gate_w.T,
        w_gate.astype(jnp.bfloat16),
        w_up.astype(jnp.bfloat16),
        w_down.astype(jnp.bfloat16),
        t_blocks=2,
        chunk=512,
    )
    return out.reshape(b, s, h)


# speedup vs baseline: 2.7915x; 1.4915x over previous
import functools

import jax
import jax.numpy as jnp
from jax import lax
from jax.experimental import pallas as pl
from jax.experimental.pallas import tpu as pltpu
from jax.experimental.pallas import tpu_sc as plsc

NC = 2
NS = 16
L = 16
NW = NC * NS

TILE = 256


def _gate_kernel(x_ref, gwt_ref, i1_ref, i2_ref, w1_ref, w2_ref):
    x = x_ref[...]
    logits = jax.lax.dot_general(
        x, gwt_ref[...], (((1,), (0,)), ((), ())),
        preferred_element_type=jnp.float32)
    t, e = logits.shape
    eidx = jax.lax.broadcasted_iota(jnp.int32, (t, e), 1)
    m1 = jnp.max(logits, axis=1, keepdims=True)
    i1 = jnp.min(jnp.where(logits == m1, eidx, e), axis=1, keepdims=True)
    lm = jnp.where(eidx == i1, -1e30, logits)
    m2 = jnp.max(lm, axis=1, keepdims=True)
    i2 = jnp.min(jnp.where(lm == m2, eidx, e), axis=1, keepdims=True)
    w1 = 1.0 / (1.0 + jnp.exp(m2 - m1))
    i1_ref[...] = i1
    i2_ref[...] = i2
    w1_ref[...] = w1
    w2_ref[...] = 1.0 - w1


def _rgather(v, idx):
    return v.at[idx].get(mode="promise_in_bounds")


def _prefix_same(ev, lanes):
    acc = jnp.zeros((L,), jnp.int32)
    for sh in range(1, L):
        shifted = _rgather(ev, jnp.maximum(lanes - sh, 0))
        acc = acc + jnp.where((lanes >= sh) & (shifted == ev), 1, 0)
    return acc


def _hist(ev, lanes):
    h = jnp.zeros((L,), jnp.int32)
    for l in range(L):
        h = h + jnp.where(lanes == ev[l], 1, 0)
    return h


def _excl_prefix_sum(v, lanes):
    acc = jnp.zeros((L,), jnp.int32)
    for sh in range(1, L):
        acc = acc + jnp.where(lanes >= sh, _rgather(v, jnp.maximum(lanes - sh, 0)), 0)
    return acc


def _route_body(i1_hbm, i2_hbm, pose_o, poso_o, texp_o, tvalid_o, hist_o,
                i1l, i2l, pel, pol, hv, allh, texpv, tvalv, *,
                n_exp, toks, maxt_pad):
    sid = lax.axis_index("s")
    tpr = toks // NS
    nv = tpr // L
    t0 = sid * tpr
    pltpu.sync_copy(i1_hbm.at[pl.ds(t0, tpr)], i1l)
    pltpu.sync_copy(i2_hbm.at[pl.ds(t0, tpr)], i2l)
    lanes = lax.iota(jnp.int32, L)

    cnte = jnp.zeros((L,), jnp.int32)
    for m in range(nv):
        sl = pl.ds(m * L, L)
        i1v = i1l[sl]
        pel[sl] = _rgather(cnte, i1v) + _prefix_same(i1v, lanes)
        cnte = cnte + _hist(i1v, lanes)
    cnto = jnp.zeros((L,), jnp.int32)
    for m in range(nv):
        sl = pl.ds(m * L, L)
        i2v = i2l[sl]
        pol[sl] = (_rgather(cnte, i2v) + _rgather(cnto, i2v)
                   + _prefix_same(i2v, lanes))
        cnto = cnto + _hist(i2v, lanes)

    hv[...] = cnte + cnto
    pltpu.sync_copy(hv, hist_o.at[sid])
    plsc.subcore_barrier()
    pltpu.sync_copy(hist_o, allh)
    total = jnp.zeros((L,), jnp.int32)
    mybase = jnp.zeros((L,), jnp.int32)
    for sp in range(NS):
        row = allh[sp]
        total = total + row
        mybase = mybase + jnp.where(sp < sid, row, 0)
    pt = ((total + (TILE - 1)) >> 8) << 8
    padoff = _excl_prefix_sum(pt, lanes)
    base_add = padoff + mybase

    for m in range(nv):
        sl = pl.ds(m * L, L)
        pel[sl] = pel[sl] + _rgather(base_add, i1l[sl])
        pol[sl] = pol[sl] + _rgather(base_add, i2l[sl])
    pltpu.sync_copy(pel, pose_o.at[pl.ds(t0, tpr)])
    pltpu.sync_copy(pol, poso_o.at[pl.ds(t0, tpr)])

    tstart = padoff >> 8
    nt = ((padoff + pt) >> 8)[n_exp - 1]
    for bb in range(maxt_pad // L):
        tv = lax.iota(jnp.int32, L) + bb * L
        tx = jnp.zeros((L,), jnp.int32)
        for e in range(1, n_exp):
            tx = tx + jnp.where(tv >= tstart[e], 1, 0)
        val = jnp.where(tv < nt, 1, 0)
        texpv[pl.ds(bb * L, L)] = jnp.where(val == 1, tx, n_exp - 1)
        tvalv[pl.ds(bb * L, L)] = val
    pltpu.sync_copy(texpv, texp_o)
    pltpu.sync_copy(tvalv, tvalid_o)


def _scatter_body(pose_hbm, poso_hbm, x_hbm, xs_hbm,
                  idx2, xloc, sem, *, tpw, ch):
    cid = lax.axis_index("c")
    sid = lax.axis_index("s")
    wid = sid * NC + cid
    t0 = wid * tpw
    for c in range(tpw // ch):
        tc = t0 + c * ch
        pltpu.sync_copy(pose_hbm.at[pl.ds(tc, ch)], idx2.at[0])
        pltpu.sync_copy(poso_hbm.at[pl.ds(tc, ch)], idx2.at[1])
        pltpu.sync_copy(x_hbm.at[pl.ds(tc, ch)], xloc)
        pltpu.async_copy(xloc, xs_hbm.at[idx2.at[0]], sem).wait()
        pltpu.async_copy(xloc, xs_hbm.at[idx2.at[1]], sem).wait()


def _combine_body(pose_hbm, poso_hbm, w1_hbm, w2_hbm, y_hbm, out_hbm,
                  pel, pol, w1l, w2l, bufa, bufb, sem, *, tpw, h):
    cid = lax.axis_index("c")
    sid = lax.axis_index("s")
    wid = sid * NC + cid
    t0 = wid * tpw
    pltpu.sync_copy(pose_hbm.at[pl.ds(t0, tpw)], pel)
    pltpu.sync_copy(poso_hbm.at[pl.ds(t0, tpw)], pol)
    pltpu.sync_copy(w1_hbm.at[pl.ds(t0, tpw)], w1l)
    pltpu.sync_copy(w2_hbm.at[pl.ds(t0, tpw)], w2l)
    for c in range(tpw // L):
        sl = pl.ds(c * L, L)
        cpa = pltpu.async_copy(y_hbm.at[pel.at[sl]], bufa, sem)
        cpb = pltpu.async_copy(y_hbm.at[pol.at[sl]], bufb, sem)
        cpa.wait()
        cpb.wait()
        w1v = w1l[sl]
        w2v = w2l[sl]

        def tbody(t, _):
            wsa = _rgather(w1v, jnp.broadcast_to(t, (L,)))
            wsb = _rgather(w2v, jnp.broadcast_to(t, (L,)))
            for k in range(h // L):
                ksl = pl.ds(k * L, L)
                bufa[t, ksl] = bufa[t, ksl] * wsa + bufb[t, ksl] * wsb
            return 0

        lax.fori_loop(0, L, tbody, 0)
        pltpu.sync_copy(bufa, out_hbm.at[pl.ds(t0 + c * L, L)])


def _mm_kernel(texp_ref, tval_ref, xs_ref, wg_ref, wu_ref, wd_ref, y_ref):
    i = pl.program_id(0)

    @pl.when(tval_ref[i] == 1)
    def _():
        xc = xs_ref[...].astype(jnp.bfloat16)
        g = jax.lax.dot_general(xc, wg_ref[0], (((1,), (0,)), ((), ())),
                                preferred_element_type=jnp.float32)
        u = jax.lax.dot_general(xc, wu_ref[0], (((1,), (0,)), ((), ())),
                                preferred_element_type=jnp.float32)
        hh = (g * jax.lax.logistic(g) * u).astype(jnp.bfloat16)
        y_ref[...] = jax.lax.dot_general(hh, wd_ref[0], (((1,), (0,)), ((), ())),
                                         preferred_element_type=jnp.float32)


def kernel(hidden_states, gate_w, w_gate, w_up, w_down):
    b, s, h = hidden_states.shape
    e_num, _, f = w_gate.shape
    t = b * s
    maxr = 2 * t + e_num * TILE
    maxt = maxr // TILE
    maxt_pad = ((maxt + L - 1) // L) * L
    tpw = t // NW

    x = hidden_states.reshape(t, h)
    i1, i2, w1, w2 = pl.pallas_call(
        _gate_kernel,
        out_shape=[
            jax.ShapeDtypeStruct((t, 1), jnp.int32),
            jax.ShapeDtypeStruct((t, 1), jnp.int32),
            jax.ShapeDtypeStruct((t, 1), jnp.float32),
            jax.ShapeDtypeStruct((t, 1), jnp.float32),
        ],
    )(x, gate_w.T)
    i1f = i1.reshape(t)
    i2f = i2.reshape(t)
    w1f = w1.reshape(t)
    w2f = w2.reshape(t)

    mesh = plsc.VectorSubcoreMesh(core_axis_name="c", subcore_axis_name="s",
                                  num_cores=NC, num_subcores=NS)
    tpr = t // NS

    route = pl.kernel(
        functools.partial(_route_body, n_exp=e_num, toks=t, maxt_pad=maxt_pad),
        out_type=[
            jax.ShapeDtypeStruct((t,), jnp.int32),
            jax.ShapeDtypeStruct((t,), jnp.int32),
            jax.ShapeDtypeStruct((maxt_pad,), jnp.int32),
            jax.ShapeDtypeStruct((maxt_pad,), jnp.int32),
            jax.ShapeDtypeStruct((NS, L), jnp.int32),
        ],
        mesh=mesh,
        scratch_types=[
            pltpu.VMEM((tpr,), jnp.int32),
            pltpu.VMEM((tpr,), jnp.int32),
            pltpu.VMEM((tpr,), jnp.int32),
            pltpu.VMEM((tpr,), jnp.int32),
            pltpu.VMEM((L,), jnp.int32),
            pltpu.VMEM((NS, L), jnp.int32),
            pltpu.VMEM((maxt_pad,), jnp.int32),
            pltpu.VMEM((maxt_pad,), jnp.int32),
        ],
    )
    pose, poso, texp, tvalid, _ = route(i1f, i2f)

    ch = 64
    scatter = pl.kernel(
        functools.partial(_scatter_body, tpw=tpw, ch=ch),
        out_type=jax.ShapeDtypeStruct((maxr, h), jnp.float32),
        mesh=mesh,
        scratch_types=[
            pltpu.VMEM((2, ch), jnp.int32),
            pltpu.VMEM((ch, h), jnp.float32),
            pltpu.SemaphoreType.DMA,
        ],
    )
    xs = scatter(pose, poso, x)

    grid_spec = pltpu.PrefetchScalarGridSpec(
        num_scalar_prefetch=2,
        grid=(maxt,),
        in_specs=[
            pl.BlockSpec((TILE, h), lambda i, texp, tval: (i, 0)),
            pl.BlockSpec((1, h, f), lambda i, texp, tval: (texp[i], 0, 0)),
            pl.BlockSpec((1, h, f), lambda i, texp, tval: (texp[i], 0, 0)),
            pl.BlockSpec((1, f, h), lambda i, texp, tval: (texp[i], 0, 0)),
        ],
        out_specs=pl.BlockSpec((TILE, h), lambda i, texp, tval: (i, 0)),
    )
    y = pl.pallas_call(
        _mm_kernel,
        grid_spec=grid_spec,
        out_shape=jax.ShapeDtypeStruct((maxr, h), jnp.float32),
    )(texp, tvalid, xs,
      w_gate.astype(jnp.bfloat16),
      w_up.astype(jnp.bfloat16),
      w_down.astype(jnp.bfloat16))

    combine = pl.kernel(
        functools.partial(_combine_body, tpw=tpw, h=h),
        out_type=jax.ShapeDtypeStruct((t, h), jnp.float32),
        mesh=mesh,
        scratch_types=[
            pltpu.VMEM((tpw,), jnp.int32),
            pltpu.VMEM((tpw,), jnp.int32),
            pltpu.VMEM((tpw,), jnp.float32),
            pltpu.VMEM((tpw,), jnp.float32),
            pltpu.VMEM((L, h), jnp.float32),
            pltpu.VMEM((L, h), jnp.float32),
            pltpu.SemaphoreType.DMA,
        ],
    )
    out = combine(pose, poso, w1f, w2f, y)
    return out.reshape(b, s, h)
